# baseline (speedup 1.0000x reference)
import functools

import jax
import jax.numpy as jnp
from jax import lax
from jax.experimental import pallas as pl
from jax.experimental.pallas import tpu as pltpu
from jax.experimental.pallas import tpu_sc as plsc

NC = 2
NS = 16
NW = NC * NS
CHUNK = 128


@functools.lru_cache(maxsize=None)
def _build(total_rows: int, emb_dim: int, vocab_rows: int):
    rows_per_w = total_rows // NW
    nchunk = rows_per_w // CHUNK
    assert rows_per_w * NW == total_rows and nchunk * CHUNK == rows_per_w

    mesh = plsc.VectorSubcoreMesh(core_axis_name="c", subcore_axis_name="s")

    @functools.partial(
        pl.kernel,
        mesh=mesh,
        compiler_params=pltpu.CompilerParams(use_tc_tiling_on_sc=False),
        out_type=jax.ShapeDtypeStruct((total_rows, emb_dim), jnp.float32),
        scratch_types=[
            pltpu.VMEM((nchunk, CHUNK), jnp.int32),
            pltpu.VMEM((CHUNK, emb_dim), jnp.float32),
            pltpu.SemaphoreType.DMA,
        ],
    )
    def gather_kernel(gidx_hbm, table_hbm, out_hbm, idx_v, rows_v, sem):
        wid = lax.axis_index("s") * NC + lax.axis_index("c")
        base = wid * rows_per_w
        pltpu.sync_copy(gidx_hbm.at[wid], idx_v)

        def body(c, carry):
            cp = pltpu.async_copy(table_hbm.at[idx_v.at[c]], rows_v, sem)
            cp.wait()
            pltpu.sync_copy(rows_v, out_hbm.at[pl.ds(base + c * CHUNK, CHUNK)])
            return carry

        lax.fori_loop(0, nchunk, body, 0)

    return gather_kernel


def kernel(inputs, tables):
    batch, num_tables = inputs.shape
    _, vocab, emb_dim = tables.shape
    total = batch * num_tables

    offsets = (jnp.arange(num_tables, dtype=jnp.int32) * vocab)[None, :]
    gidx = (inputs + offsets).reshape(NW, total // NW // CHUNK, CHUNK)
    flat_tables = tables.reshape(num_tables * vocab, emb_dim)

    out = _build(total, emb_dim, num_tables * vocab)(gidx, flat_tables)
    return out.reshape(batch, num_tables, emb_dim)
